# baseline (speedup 1.0000x reference)
import functools

import jax
import jax.numpy as jnp
from jax import lax
from jax.experimental import pallas as pl
from jax.experimental.pallas import tpu as pltpu
from jax.experimental.pallas import tpu_sc as plsc

Q = 1024
N = 100000
D = 16
K = 16
L = 16
NC, NS = 2, 16
NW = NC * NS
QPW = Q // NW
NPAD = 100352
CHUNK = 7168
NCHUNK = NPAD // CHUNK
NGROUPS = CHUNK // L
BPG = 8
NBLOCKS = NGROUPS // BPG
BUFCAP = 112
BUFSZ = 640
NBLK = 512
import numpy as np

INF = np.float32(3.0e38)


def _tc_d2_body(xq_ref, xt_ref, o_ref):
    q = xq_ref[...]
    t = xt_ref[...]
    dot = lax.dot_general(q, t, (((1,), (1,)), ((), ())),
                          preferred_element_type=jnp.float32)
    q2 = jnp.sum(q * q, axis=1, keepdims=True)
    t2 = jnp.sum(t * t, axis=1)[None, :]
    o_ref[...] = q2 - 2.0 * dot + t2


def _compute_d2(X_test, Xp):
    grid = (NPAD // NBLK,)
    return pl.pallas_call(
        _tc_d2_body,
        grid=grid,
        in_specs=[
            pl.BlockSpec((Q, D), lambda i: (0, 0)),
            pl.BlockSpec((NBLK, D), lambda i: (i, 0)),
        ],
        out_specs=pl.BlockSpec((Q, NBLK), lambda i: (0, i)),
        out_shape=jax.ShapeDtypeStruct((Q, NPAD), jnp.float32),
    )(X_test, Xp)


def _merge_sorted(ad, av, bd, bv):
    bdr = lax.rev(bd, (0,))
    bvr = lax.rev(bv, (0,))
    take = ad <= bdr
    nd = jnp.where(take, ad, bdr)
    nv = jnp.where(take, av, bvr)
    sd, sv = plsc.sort_key_val(nd, nv)
    return sd, sv


def _sc_topk_vote(d2, yp):
    mesh = plsc.VectorSubcoreMesh(core_axis_name="c", subcore_axis_name="s",
                                  num_cores=NC, num_subcores=NS)

    @functools.partial(
        pl.kernel,
        out_type=jax.ShapeDtypeStruct((Q,), jnp.int32),
        mesh=mesh,
        scratch_types=[
            pltpu.VMEM((2, CHUNK + L), jnp.float32),
            pltpu.VMEM((BUFSZ,), jnp.int32),
            pltpu.VMEM((NPAD,), jnp.int32),
            pltpu.VMEM((L,), jnp.float32),
            pltpu.VMEM((L,), jnp.int32),
            pltpu.SMEM((QPW,), jnp.int32),
            pltpu.SMEM((2,), jnp.int32),
            pltpu.SMEM((2,), jnp.float32),
            pltpu.VMEM((QPW,), jnp.int32),
            pltpu.SemaphoreType.DMA((2,)),
        ],
        compiler_params=pltpu.CompilerParams(needs_layout_passes=False),
    )
    def kern(d2_hbm, yp_hbm, y_hbm, chunkbuf, idxbuf, yp_v, topd_v, topgi_v,
             ybuf_s, scnt, sthr, ybuf, csems):
        wid = lax.axis_index("s") * NC + lax.axis_index("c")
        q0 = wid * QPW
        iota = lax.iota(jnp.int32, L)
        inf_vec = jnp.full((L,), INF, jnp.float32)

        chunkbuf[0, pl.ds(CHUNK, L)] = inf_vec
        chunkbuf[1, pl.ds(CHUNK, L)] = inf_vec
        pltpu.sync_copy(yp_hbm, yp_v)

        def start_dma(t, slot):
            qg = q0 + lax.div(t, NCHUNK)
            cb = lax.rem(t, NCHUNK) * CHUNK
            return pltpu.make_async_copy(
                d2_hbm.at[qg, pl.ds(cb, CHUNK)],
                chunkbuf.at[slot, pl.ds(0, CHUNK)],
                csems.at[slot])

        start_dma(0, 0).start()

        T = QPW * NCHUNK

        def tbody(t, _c):
            parity = lax.rem(t, 2)
            c = lax.rem(t, NCHUNK)
            cbase = c * CHUNK

            start_dma(t, parity).wait()

            @pl.when(t + 1 < T)
            def _():
                start_dma(t + 1, 1 - parity).start()

            @pl.when(c == 0)
            def _():
                topd_v[...] = inf_vec
                topgi_v[...] = jnp.zeros((L,), jnp.int32)
                sthr[0] = jnp.float32(INF)
                scnt[0] = 0

            def consolidate():
                cnt = scnt[0]
                idxbuf[pl.ds(cnt, L)] = jnp.full((L,), CHUNK, jnp.int32)
                ng = lax.shift_right_logical(cnt + (L - 1), 4)
                pvec = jnp.full((L,), parity, jnp.int32)

                def mbody(i, mcarry):
                    topd, topgi = mcarry
                    idxv = idxbuf[pl.ds(i * L, L)]
                    d2v = plsc.load_gather(chunkbuf, [pvec, idxv])
                    giv = idxv + cbase
                    sd, sg = plsc.sort_key_val(d2v, giv)
                    return _merge_sorted(topd, topgi, sd, sg)

                topd, topgi = lax.fori_loop(
                    0, ng, mbody, (topd_v[...], topgi_v[...]))
                topd_v[...] = topd
                topgi_v[...] = topgi
                sthr[0] = topd[L - 1]
                scnt[0] = 0

            def bbody(b, _b):
                gbase = b * BPG

                mins = [chunkbuf[parity, pl.ds((gbase + j) * L, L)]
                        for j in range(BPG)]
                while len(mins) > 1:
                    mins = [jnp.minimum(mins[2 * i], mins[2 * i + 1])
                            for i in range(len(mins) // 2)]
                bmin = jnp.min(mins[0])

                @pl.when(bmin < sthr[0])
                def _():
                    thr = jnp.full((L,), sthr[0])
                    cnt = scnt[0]
                    for j in range(BPG):
                        cand = chunkbuf[parity, pl.ds((gbase + j) * L, L)]
                        m = cand < thr
                        loc = iota + (gbase + j) * L
                        idxbuf[pl.ds(cnt, L)] = jnp.where(m, loc, CHUNK)
                        npass = jnp.sum(m.astype(jnp.int32))
                        cnt = cnt + jnp.where(npass > 0, L, 0)
                    scnt[0] = cnt

                    @pl.when(cnt >= BUFCAP)
                    def _():
                        consolidate()

                return 0

            lax.fori_loop(0, NBLOCKS, bbody, 0)

            consolidate()

            @pl.when(c == NCHUNK - 1)
            def _():
                labs = plsc.load_gather(yp_v, [topgi_v[...]])
                cntv = jnp.zeros((L,), jnp.int32)
                for j in range(L):
                    cntv = cntv + jnp.where(labs == labs[j], 1, 0)
                score = cntv * 128 + (127 - labs)
                best = jnp.max(score)
                win = 127 - lax.rem(best, 128)
                ybuf_s[lax.div(t, NCHUNK)] = win

            return 0

        lax.fori_loop(0, T, tbody, 0)
        for v in range(QPW // L):
            acc = jnp.zeros((L,), jnp.int32)
            for i in range(L):
                acc = jnp.where(iota == i, ybuf_s[v * L + i], acc)
            ybuf[pl.ds(v * L, L)] = acc
        pltpu.sync_copy(ybuf, y_hbm.at[pl.ds(q0, QPW)])

    return kern(d2, yp)


def kernel(X_test, X_train, y_train):
    Xp = jnp.concatenate(
        [X_train, jnp.full((NPAD - N, D), 1.0e4, jnp.float32)], axis=0)
    yp = jnp.concatenate(
        [y_train, jnp.zeros((NPAD - N,), jnp.int32)], axis=0)
    d2 = _compute_d2(X_test, Xp)
    return _sc_topk_vote(d2, yp)


# speedup vs baseline: 3.4109x; 3.4109x over previous
import functools

import jax
import jax.numpy as jnp
import numpy as np
from jax import lax
from jax.experimental import pallas as pl
from jax.experimental.pallas import tpu as pltpu
from jax.experimental.pallas import tpu_sc as plsc

Q = 1024
N = 100000
D = 16
L = 16
NC, NS = 2, 16
NW = NC * NS
QPW = Q // NW
NPAD = 100352
SEG = 128
NSEG = NPAD // SEG
GPAD = 896
NGV = GPAD // L
NBLK = 512
INF = np.float32(3.0e38)


def _tc_d2_body(xq_ref, xt_ref, o_ref):
    q = xq_ref[...]
    t = xt_ref[...]
    dot = lax.dot_general(q, t, (((1,), (1,)), ((), ())),
                          preferred_element_type=jnp.float32)
    q2 = jnp.sum(q * q, axis=1, keepdims=True)
    t2 = jnp.sum(t * t, axis=1)[None, :]
    o_ref[...] = q2 - 2.0 * dot + t2


def _compute_d2(X_test, Xp):
    grid = (NPAD // NBLK,)
    return pl.pallas_call(
        _tc_d2_body,
        grid=grid,
        in_specs=[
            pl.BlockSpec((Q, D), lambda i: (0, 0)),
            pl.BlockSpec((NBLK, D), lambda i: (i, 0)),
        ],
        out_specs=pl.BlockSpec((Q, NBLK), lambda i: (0, i)),
        out_shape=jax.ShapeDtypeStruct((Q, NPAD), jnp.float32),
    )(X_test, Xp)


GQR = 32


def _tc_gmin_body(d3_ref, m_ref):
    m_ref[...] = jnp.min(d3_ref[...], axis=2)


def _compute_gmin(d2):
    d3 = d2.reshape(Q, NSEG, SEG)
    grid = (Q // GQR,)
    gm = pl.pallas_call(
        _tc_gmin_body,
        grid=grid,
        in_specs=[pl.BlockSpec((GQR, NSEG, SEG), lambda i: (i, 0, 0))],
        out_specs=pl.BlockSpec((GQR, NSEG), lambda i: (i, 0)),
        out_shape=jax.ShapeDtypeStruct((Q, NSEG), jnp.float32),
    )(d3)
    return jnp.pad(gm, ((0, 0), (0, GPAD - NSEG)), constant_values=INF)


def _merge_sorted(ad, av, bd, bv):
    bdr = lax.rev(bd, (0,))
    bvr = lax.rev(bv, (0,))
    take = ad <= bdr
    nd = jnp.where(take, ad, bdr)
    nv = jnp.where(take, av, bvr)
    sd, sv = plsc.sort_key_val(nd, nv)
    return sd, sv


def _sc_topk_vote(d2, gmin, yp):
    mesh = plsc.VectorSubcoreMesh(core_axis_name="c", subcore_axis_name="s",
                                  num_cores=NC, num_subcores=NS)

    @functools.partial(
        pl.kernel,
        out_type=jax.ShapeDtypeStruct((Q,), jnp.int32),
        mesh=mesh,
        scratch_types=[
            pltpu.VMEM((2 * GPAD,), jnp.float32),
            pltpu.VMEM((L * SEG,), jnp.float32),
            pltpu.VMEM((NPAD,), jnp.int32),
            pltpu.VMEM((L,), jnp.float32),
            pltpu.VMEM((L,), jnp.int32),
            pltpu.SMEM((QPW,), jnp.int32),
            pltpu.SMEM((L,), jnp.int32),
            pltpu.SMEM((2,), jnp.float32),
            pltpu.VMEM((QPW,), jnp.int32),
            pltpu.SemaphoreType.DMA((2,)),
            pltpu.SemaphoreType.DMA,
        ],
        compiler_params=pltpu.CompilerParams(needs_layout_passes=False),
    )
    def kern(d2_hbm, gmin_hbm, yp_hbm, y_hbm, gbuf, segbuf, yp_v,
             topd_v, topgi_v, ybuf_s, ssid, sthr, ybuf, gsems, ssem):
        wid = lax.axis_index("s") * NC + lax.axis_index("c")
        q0 = wid * QPW
        iota = lax.iota(jnp.int32, L)
        inf_vec = jnp.full((L,), INF, jnp.float32)

        pltpu.sync_copy(yp_hbm, yp_v)

        def g_dma(qi, slot):
            return pltpu.make_async_copy(
                gmin_hbm.at[q0 + qi],
                gbuf.at[pl.ds(slot * GPAD, GPAD)],
                gsems.at[slot])

        g_dma(0, 0).start()

        def s_dma(qg, sid, j):
            return pltpu.make_async_copy(
                d2_hbm.at[qg, pl.ds(sid * SEG, SEG)],
                segbuf.at[pl.ds(j * SEG, SEG)],
                ssem)

        def qbody(qi, _c):
            parity = lax.rem(qi, 2)
            qg = q0 + qi

            g_dma(qi, parity).wait()

            @pl.when(qi + 1 < QPW)
            def _():
                g_dma(qi + 1, 1 - parity).start()

            topd_v[...] = inf_vec
            topgi_v[...] = jnp.zeros((L,), jnp.int32)
            sthr[1] = jnp.float32(INF)
            gbase = parity * GPAD

            def abody(v, _a):
                gv = gbuf[pl.ds(gbase + v * L, L)]
                gm = jnp.min(gv)

                @pl.when(gm < sthr[1])
                def _():
                    sm, sid = plsc.sort_key_val(gv, iota + v * L)
                    nd, ni = _merge_sorted(topd_v[...], topgi_v[...], sm, sid)
                    topd_v[...] = nd
                    topgi_v[...] = ni
                    sthr[1] = nd[L - 1]

                return 0

            lax.fori_loop(0, NGV, abody, 0)

            sidv = topgi_v[...]
            for j in range(L):
                ssid[j] = sidv[j]

            def fire(j, _f):
                s_dma(qg, ssid[j], j).start()
                return 0

            lax.fori_loop(0, L, fire, 0)

            topd_v[...] = inf_vec
            topgi_v[...] = jnp.zeros((L,), jnp.int32)
            sthr[0] = jnp.float32(INF)

            def drain(j, _d):
                sid = ssid[j]
                s_dma(qg, sid, j).wait()
                for sg in range(SEG // L):
                    cv = segbuf[pl.ds(j * SEG + sg * L, L)]
                    cm = jnp.min(cv)

                    @pl.when(cm < sthr[0])
                    def _(sg=sg):
                        gidx = iota + (sid * SEG + sg * L)
                        sd, sgi = plsc.sort_key_val(cv, gidx)
                        nd, ngi = _merge_sorted(
                            topd_v[...], topgi_v[...], sd, sgi)
                        topd_v[...] = nd
                        topgi_v[...] = ngi
                        sthr[0] = nd[L - 1]

                return 0

            lax.fori_loop(0, L, drain, 0)

            labs = plsc.load_gather(yp_v, [topgi_v[...]])
            cntv = jnp.zeros((L,), jnp.int32)
            for j in range(L):
                cntv = cntv + jnp.where(labs == labs[j], 1, 0)
            score = cntv * 128 + (127 - labs)
            best = jnp.max(score)
            win = 127 - lax.rem(best, 128)
            ybuf_s[qi] = win

            return 0

        lax.fori_loop(0, QPW, qbody, 0)
        for v in range(QPW // L):
            acc = jnp.zeros((L,), jnp.int32)
            for i in range(L):
                acc = jnp.where(iota == i, ybuf_s[v * L + i], acc)
            ybuf[pl.ds(v * L, L)] = acc
        pltpu.sync_copy(ybuf, y_hbm.at[pl.ds(q0, QPW)])

    return kern(d2, gmin, yp)


def kernel(X_test, X_train, y_train):
    Xp = jnp.concatenate(
        [X_train, jnp.full((NPAD - N, D), 1.0e4, jnp.float32)], axis=0)
    yp = jnp.concatenate(
        [y_train, jnp.zeros((NPAD - N,), jnp.int32)], axis=0)
    d2 = _compute_d2(X_test, Xp)
    gmin = _compute_gmin(d2)
    return _sc_topk_vote(d2, gmin, yp)
